# baseline (speedup 1.0000x reference)
import functools

import jax
import jax.numpy as jnp
from jax import lax
from jax.experimental import pallas as pl
from jax.experimental.pallas import tpu as pltpu
from jax.experimental.pallas import tpu_sc as plsc

_LANES = 16
_GRP = 8


def _rsqrt_newton(x):
    i = plsc.bitcast(x, jnp.int32)
    i = jnp.int32(0x5F3759DF) - lax.shift_right_logical(i, 1)
    y = plsc.bitcast(i, jnp.float32)
    half_x = x * jnp.float32(0.5)
    for _ in range(2):
        y = y * (jnp.float32(1.5) - half_x * y * y)
    return y


@functools.lru_cache(maxsize=None)
def _build(batch, dim):
    info = plsc.get_sparse_core_info()
    num_workers = info.num_cores * info.num_subcores
    bpw = batch // num_workers
    ngroups = bpw // _LANES
    nvec = dim // _LANES
    mesh = plsc.VectorSubcoreMesh(core_axis_name="c", subcore_axis_name="s")

    @functools.partial(
        pl.kernel,
        mesh=mesh,
        compiler_params=pltpu.CompilerParams(needs_layout_passes=False),
        out_type=jax.ShapeDtypeStruct((batch,), jnp.float32),
        scratch_types=[
            pltpu.VMEM((bpw,), jnp.int32),
            pltpu.VMEM((bpw,), jnp.int32),
            pltpu.VMEM((bpw,), jnp.int32),
            [pltpu.VMEM((_LANES, _GRP, dim), jnp.float32) for _ in range(2)],
            [pltpu.VMEM((_LANES, 2 * dim), jnp.float32) for _ in range(2)],
            [pltpu.VMEM((_LANES, _GRP, dim), jnp.float32) for _ in range(2)],
            pltpu.VMEM((bpw,), jnp.float32),
            [pltpu.SemaphoreType.DMA for _ in range(2)],
            [pltpu.SemaphoreType.DMA for _ in range(2)],
            [pltpu.SemaphoreType.DMA for _ in range(2)],
        ],
    )
    def k(src_hbm, pred_hbm, tail_hbm, ev_hbm, er_hbm, out_hbm,
          si_v, pi_v, ti_v, sbufs, rbufs, tbufs, sc_v, sems, tsems, rsems):
        wid = lax.axis_index("s") * info.num_cores + lax.axis_index("c")
        base = wid * bpw
        pltpu.sync_copy(src_hbm.at[pl.ds(base, bpw)], si_v)
        pltpu.sync_copy(pred_hbm.at[pl.ds(base, bpw)], pi_v)
        pltpu.sync_copy(tail_hbm.at[pl.ds(base, bpw)], ti_v)

        iota = lax.iota(jnp.int32, _LANES)
        zero16 = jnp.zeros((_LANES,), jnp.int32)

        def fire(g, b):
            gsl = pl.ds(g * _LANES, _LANES)
            pltpu.async_copy(er_hbm.at[pi_v.at[gsl]], rbufs[b], rsems[b])
            svec = si_v[gsl]
            tvec = ti_v[gsl]
            for u in range(_LANES):
                m = iota == u
                cs = lax.reduce_sum_p.bind(jnp.where(m, svec, zero16), axes=(0,))
                ct = lax.reduce_sum_p.bind(jnp.where(m, tvec, zero16), axes=(0,))
                so = pl.multiple_of(cs & jnp.int32(~7), _GRP)
                to = pl.multiple_of(ct & jnp.int32(~7), _GRP)
                pltpu.async_copy(ev_hbm.at[pl.ds(so, _GRP), :],
                                 sbufs[b].at[u], sems[b])
                pltpu.async_copy(ev_hbm.at[pl.ds(to, _GRP), :],
                                 tbufs[b].at[u], tsems[b])

        def drain(b):
            proto = ev_hbm.at[pl.ds(0, _GRP), :]
            for u in range(_LANES):
                pltpu.make_async_copy(proto, sbufs[b].at[u], sems[b]).wait()
                pltpu.make_async_copy(proto, tbufs[b].at[u], tsems[b]).wait()
            rproto = er_hbm.at[pl.ds(0, _LANES), :]
            pltpu.make_async_copy(rproto, rbufs[b], rsems[b]).wait()

        def compute(g, b):
            gsl = pl.ds(g * _LANES, _LANES)
            svec = si_v[gsl]
            tvec = ti_v[gsl]
            s_v, r_v, t_v = sbufs[b], rbufs[b], tbufs[b]
            scores = jnp.zeros((_LANES,), jnp.float32)
            for u in range(_LANES):
                m = iota == u
                ps = lax.reduce_sum_p.bind(
                    jnp.where(m, svec & 7, zero16), axes=(0,))
                pt = lax.reduce_sum_p.bind(
                    jnp.where(m, tvec & 7, zero16), axes=(0,))
                s = [s_v[u, ps, pl.ds(v * _LANES, _LANES)] for v in range(nvec)]
                t = [t_v[u, pt, pl.ds(v * _LANES, _LANES)] for v in range(nvec)]
                ssv = s[0] * s[0]
                ttv = t[0] * t[0]
                for v in range(1, nvec):
                    ssv = ssv + s[v] * s[v]
                    ttv = ttv + t[v] * t[v]
                ss = lax.reduce_sum_p.bind(ssv, axes=(0,))
                tt = lax.reduce_sum_p.bind(ttv, axes=(0,))
                rs = _rsqrt_newton(jnp.broadcast_to(ss, (_LANES,)))
                rt = _rsqrt_newton(jnp.broadcast_to(tt, (_LANES,)))
                r = [r_v[u, pl.ds(v * _LANES, _LANES)] for v in range(nvec)]
                a = jnp.abs(s[0] * rs + r[0] - t[0] * rt)
                for v in range(1, nvec):
                    a = a + jnp.abs(s[v] * rs + r[v] - t[v] * rt)
                val = -lax.reduce_sum_p.bind(a, axes=(0,))
                scores = jnp.where(m, jnp.broadcast_to(val, (_LANES,)), scores)
            sc_v[gsl] = scores

        fire(0, 0)

        def pair(gp, _):
            g0 = gp * 2
            fire(g0 + 1, 1)
            drain(0)
            compute(g0, 0)

            @pl.when(g0 + 2 < ngroups)
            def _():
                fire(g0 + 2, 0)

            drain(1)
            compute(g0 + 1, 1)
            return 0

        lax.fori_loop(0, ngroups // 2, pair, 0)
        pltpu.sync_copy(sc_v, out_hbm.at[pl.ds(base, bpw)])

    return k


def kernel(src, pred, tail, E_v_weight, E_r_weight):
    batch = src.shape[0]
    dim = E_v_weight.shape[1]
    k = _build(batch, dim)
    er_p = jnp.pad(E_r_weight, ((0, 0), (0, dim)))
    out = k(src.astype(jnp.int32), pred.astype(jnp.int32),
            tail.astype(jnp.int32), E_v_weight, er_p)
    return out.reshape(batch, 1)


# speedup vs baseline: 1.3894x; 1.3894x over previous
import functools

import jax
import jax.numpy as jnp
from jax import lax
from jax.experimental import pallas as pl
from jax.experimental.pallas import tpu as pltpu
from jax.experimental.pallas import tpu_sc as plsc

_LANES = 16
_GRP = 8


def _rsqrt_newton(x):
    i = plsc.bitcast(x, jnp.int32)
    i = jnp.int32(0x5F3759DF) - lax.shift_right_logical(i, 1)
    y = plsc.bitcast(i, jnp.float32)
    half_x = x * jnp.float32(0.5)
    for _ in range(2):
        y = y * (jnp.float32(1.5) - half_x * y * y)
    return y


@functools.lru_cache(maxsize=None)
def _build(batch, dim):
    info = plsc.get_sparse_core_info()
    num_workers = info.num_cores * info.num_subcores
    bpw = batch // num_workers
    ngroups = bpw // _LANES
    nvec = dim // _LANES
    mesh = plsc.VectorSubcoreMesh(core_axis_name="c", subcore_axis_name="s")

    @functools.partial(
        pl.kernel,
        mesh=mesh,
        compiler_params=pltpu.CompilerParams(needs_layout_passes=False),
        out_type=jax.ShapeDtypeStruct((batch,), jnp.float32),
        scratch_types=[
            pltpu.VMEM((bpw,), jnp.int32),
            pltpu.VMEM((bpw,), jnp.int32),
            pltpu.VMEM((bpw,), jnp.int32),
            [pltpu.VMEM((_LANES, _GRP, dim), jnp.float32) for _ in range(2)],
            [pltpu.VMEM((_LANES, 2 * dim), jnp.float32) for _ in range(2)],
            [pltpu.VMEM((_LANES, _GRP, dim), jnp.float32) for _ in range(2)],
            pltpu.VMEM((bpw,), jnp.float32),
            [pltpu.SemaphoreType.DMA for _ in range(2)],
            [pltpu.SemaphoreType.DMA for _ in range(2)],
            [pltpu.SemaphoreType.DMA for _ in range(2)],
        ],
    )
    def k(src_hbm, pred_hbm, tail_hbm, ev_hbm, er_hbm, out_hbm,
          si_v, pi_v, ti_v, sbufs, rbufs, tbufs, sc_v, sems, tsems, rsems):
        wid = lax.axis_index("s") * info.num_cores + lax.axis_index("c")
        base = wid * bpw
        pltpu.sync_copy(src_hbm.at[pl.ds(base, bpw)], si_v)
        pltpu.sync_copy(pred_hbm.at[pl.ds(base, bpw)], pi_v)
        pltpu.sync_copy(tail_hbm.at[pl.ds(base, bpw)], ti_v)

        iota = lax.iota(jnp.int32, _LANES)
        zero16 = jnp.zeros((_LANES,), jnp.int32)

        def fire(g, b):
            gsl = pl.ds(g * _LANES, _LANES)
            pltpu.async_copy(er_hbm.at[pi_v.at[gsl]], rbufs[b], rsems[b])
            svec = si_v[gsl]
            tvec = ti_v[gsl]
            for u in range(_LANES):
                m = iota == u
                cs = lax.reduce_sum_p.bind(jnp.where(m, svec, zero16), axes=(0,))
                ct = lax.reduce_sum_p.bind(jnp.where(m, tvec, zero16), axes=(0,))
                so = lax.shift_right_logical(cs, 3)
                to = lax.shift_right_logical(ct, 3)
                pltpu.async_copy(ev_hbm.at[so], sbufs[b].at[u], sems[b])
                pltpu.async_copy(ev_hbm.at[to], tbufs[b].at[u], tsems[b])

        def drain(b):
            proto = ev_hbm.at[0]
            for u in range(_LANES):
                pltpu.make_async_copy(proto, sbufs[b].at[u], sems[b]).wait()
                pltpu.make_async_copy(proto, tbufs[b].at[u], tsems[b]).wait()
            rproto = er_hbm.at[pl.ds(0, _LANES), :]
            pltpu.make_async_copy(rproto, rbufs[b], rsems[b]).wait()

        def compute(g, b):
            gsl = pl.ds(g * _LANES, _LANES)
            svec = si_v[gsl]
            tvec = ti_v[gsl]
            s_v, r_v, t_v = sbufs[b], rbufs[b], tbufs[b]
            scores = jnp.zeros((_LANES,), jnp.float32)
            for u in range(_LANES):
                m = iota == u
                ps = lax.reduce_sum_p.bind(
                    jnp.where(m, svec & 7, zero16), axes=(0,))
                pt = lax.reduce_sum_p.bind(
                    jnp.where(m, tvec & 7, zero16), axes=(0,))
                s = [s_v[u, ps, pl.ds(v * _LANES, _LANES)] for v in range(nvec)]
                t = [t_v[u, pt, pl.ds(v * _LANES, _LANES)] for v in range(nvec)]
                ssv = s[0] * s[0]
                ttv = t[0] * t[0]
                for v in range(1, nvec):
                    ssv = ssv + s[v] * s[v]
                    ttv = ttv + t[v] * t[v]
                ss = lax.reduce_sum_p.bind(ssv, axes=(0,))
                tt = lax.reduce_sum_p.bind(ttv, axes=(0,))
                rs = _rsqrt_newton(jnp.broadcast_to(ss, (_LANES,)))
                rt = _rsqrt_newton(jnp.broadcast_to(tt, (_LANES,)))
                r = [r_v[u, pl.ds(v * _LANES, _LANES)] for v in range(nvec)]
                a = jnp.abs(s[0] * rs + r[0] - t[0] * rt)
                for v in range(1, nvec):
                    a = a + jnp.abs(s[v] * rs + r[v] - t[v] * rt)
                val = -lax.reduce_sum_p.bind(a, axes=(0,))
                scores = jnp.where(m, jnp.broadcast_to(val, (_LANES,)), scores)
            sc_v[gsl] = scores

        fire(0, 0)

        def pair(gp, _):
            g0 = gp * 2
            fire(g0 + 1, 1)
            drain(0)
            compute(g0, 0)

            @pl.when(g0 + 2 < ngroups)
            def _():
                fire(g0 + 2, 0)

            drain(1)
            compute(g0 + 1, 1)
            return 0

        lax.fori_loop(0, ngroups // 2, pair, 0)
        pltpu.sync_copy(sc_v, out_hbm.at[pl.ds(base, bpw)])

    return k


def kernel(src, pred, tail, E_v_weight, E_r_weight):
    batch = src.shape[0]
    dim = E_v_weight.shape[1]
    k = _build(batch, dim)
    er_p = jnp.pad(E_r_weight, ((0, 0), (0, dim)))
    ev3 = E_v_weight.reshape(-1, _GRP, dim)
    out = k(src.astype(jnp.int32), pred.astype(jnp.int32),
            tail.astype(jnp.int32), ev3, er_p)
    return out.reshape(batch, 1)
